# baseline (speedup 1.0000x reference)
import functools

import jax
import jax.numpy as jnp
from jax import lax
from jax.experimental import pallas as pl
from jax.experimental.pallas import tpu as pltpu
from jax.experimental.pallas import tpu_sc as plsc

EPS = 1e-08
N = 100000
C = 128
G_MAX = 64
NC = 2
NS = 16
NW = NC * NS
RPW = N // NW
RPW_PAD = 3144
CHUNK = 125
CHW = CHUNK * C
NCHUNK = RPW // CHUNK
ENT = G_MAX * C
EPW = ENT // NS
CG = C // 16
LANES = 16

_mesh = plsc.VectorSubcoreMesh(core_axis_name="c", subcore_axis_name="s")


def _phase1_body(tflat_hbm, batch_hbm, pmax_hbm, pidx_hbm,
                 batch_v, buf, sbuf, accmax, accidx, regm, regi, segs,
                 sem, sem_a, sem_b):
    cid = lax.axis_index("c")
    sid = lax.axis_index("s")
    w = cid * NS + sid
    row0 = w * RPW
    sid2 = sid * 2 * CHW

    pltpu.sync_copy(batch_hbm.at[pl.ds(w * RPW_PAD, RPW_PAD)], batch_v)

    def init_body(k, _):
        accmax[pl.ds(k * LANES, LANES)] = jnp.full((LANES,), -1.0, jnp.float32)
        accidx[pl.ds(k * LANES, LANES)] = jnp.full((LANES,), N - 1, jnp.int32)
        return 0

    lax.fori_loop(0, ENT // LANES, init_body, 0)
    for cg in range(CG):
        regm[pl.ds(cg * LANES, LANES)] = jnp.full((LANES,), -1.0, jnp.float32)
        regi[pl.ds(cg * LANES, LANES)] = jnp.full((LANES,), N - 1, jnp.int32)
    segs[0] = batch_v[pl.ds(0, LANES)][0]

    def dma_a(ch):
        return pltpu.make_async_copy(
            tflat_hbm.at[pl.ds((row0 + ch * CHUNK) * C, CHW)],
            sbuf.at[pl.ds(sid2 + lax.rem(ch, 2) * CHW, CHW)],
            sem_a)

    def dma_b(ch):
        return pltpu.make_async_copy(
            sbuf.at[pl.ds(sid2 + lax.rem(ch, 2) * CHW, CHW)],
            buf.at[pl.ds(lax.rem(ch, 2) * CHW, CHW)],
            sem_b)

    dma_a(0).start()
    dma_a(1).start()
    dma_a(0).wait()
    dma_b(0).start()

    def flush(seg, vm, vi):
        base = seg * C
        for cg in range(CG):
            cur = accmax[pl.ds(base + cg * LANES, LANES)]
            curi = accidx[pl.ds(base + cg * LANES, LANES)]
            m = vm[cg] > cur
            accmax[pl.ds(base + cg * LANES, LANES)] = jnp.where(m, vm[cg], cur)
            accidx[pl.ds(base + cg * LANES, LANES)] = jnp.where(m, vi[cg], curi)

    def chunk_body(ch, _):
        dma_b(ch).wait()

        @pl.when(ch + 2 < NCHUNK)
        def _():
            dma_a(ch + 2).start()

        @pl.when(ch + 1 < NCHUNK)
        def _():
            dma_a(ch + 1).wait()
            dma_b(ch + 1).start()

        boff = lax.rem(ch, 2) * CHW
        cur_seg = segs[0]
        first = batch_v[pl.ds(ch * CHUNK, LANES)][0]
        last = batch_v[pl.ds(ch * CHUNK + CHUNK - LANES, LANES)][LANES - 1]
        uniform = jnp.logical_and(first == last, first == cur_seg)

        @pl.when(uniform)
        def _():
            vm = [regm[pl.ds(cg * LANES, LANES)] for cg in range(CG)]
            vi = [regi[pl.ds(cg * LANES, LANES)] for cg in range(CG)]

            def row_body(r, rc):
                rvm = list(rc[:CG])
                rvi = list(rc[CG:])
                i0 = r * 2
                i1 = r * 2 + 1
                ridx0 = jnp.full((LANES,), row0 + ch * CHUNK + i0, jnp.int32)
                ridx1 = jnp.full((LANES,), row0 + ch * CHUNK + i1, jnp.int32)
                for cg in range(CG):
                    v0 = jnp.abs(buf[pl.ds(boff + i0 * C + cg * LANES, LANES)])
                    v1 = jnp.abs(buf[pl.ds(boff + i1 * C + cg * LANES, LANES)])
                    m01 = v1 > v0
                    vp = jnp.maximum(v0, v1)
                    ip = jnp.where(m01, ridx1, ridx0)
                    m = vp > rvm[cg]
                    rvm[cg] = jnp.maximum(rvm[cg], vp)
                    rvi[cg] = jnp.where(m, ip, rvi[cg])
                return tuple(rvm + rvi)

            res = lax.fori_loop(0, CHUNK // 2, row_body, tuple(vm + vi))

            ilast = CHUNK - 1
            rvm = list(res[:CG])
            rvi = list(res[CG:])
            ridx = jnp.full((LANES,), row0 + ch * CHUNK + ilast, jnp.int32)
            for cg in range(CG):
                v = jnp.abs(buf[pl.ds(boff + ilast * C + cg * LANES, LANES)])
                m = v > rvm[cg]
                rvm[cg] = jnp.maximum(rvm[cg], v)
                rvi[cg] = jnp.where(m, ridx, rvi[cg])
            res = tuple(rvm + rvi)
            for cg in range(CG):
                regm[pl.ds(cg * LANES, LANES)] = res[cg]
                regi[pl.ds(cg * LANES, LANES)] = res[CG + cg]

        @pl.when(jnp.logical_not(uniform))
        def _():
            vm = [regm[pl.ds(cg * LANES, LANES)] for cg in range(CG)]
            vi = [regi[pl.ds(cg * LANES, LANES)] for cg in range(CG)]
            flush(cur_seg, vm, vi)

            def row_body(i, _):
                seg = batch_v[pl.ds(ch * CHUNK + i, LANES)][0]
                base = seg * C
                ridx = jnp.full((LANES,), row0 + ch * CHUNK + i, jnp.int32)
                for cg in range(CG):
                    v = jnp.abs(buf[pl.ds(boff + i * C + cg * LANES, LANES)])
                    cur = accmax[pl.ds(base + cg * LANES, LANES)]
                    curi = accidx[pl.ds(base + cg * LANES, LANES)]
                    m = v > cur
                    accmax[pl.ds(base + cg * LANES, LANES)] = jnp.where(m, v, cur)
                    accidx[pl.ds(base + cg * LANES, LANES)] = jnp.where(
                        m, ridx, curi)
                return 0

            lax.fori_loop(0, CHUNK, row_body, 0)
            nbase = last * C
            for cg in range(CG):
                regm[pl.ds(cg * LANES, LANES)] = accmax[
                    pl.ds(nbase + cg * LANES, LANES)]
                regi[pl.ds(cg * LANES, LANES)] = accidx[
                    pl.ds(nbase + cg * LANES, LANES)]
            segs[0] = last

        return 0

    lax.fori_loop(0, NCHUNK, chunk_body, 0)
    flush(segs[0],
          [regm[pl.ds(cg * LANES, LANES)] for cg in range(CG)],
          [regi[pl.ds(cg * LANES, LANES)] for cg in range(CG)])

    for s in range(NS):
        pltpu.async_copy(
            accmax.at[pl.ds(s * EPW, EPW)],
            pmax_hbm.at[pl.ds(s * NW * EPW + w * EPW, EPW)], sem)
        pltpu.async_copy(
            accidx.at[pl.ds(s * EPW, EPW)],
            pidx_hbm.at[pl.ds(s * NW * EPW + w * EPW, EPW)], sem)
    for s in range(NS):
        pltpu.make_async_copy(
            accmax.at[pl.ds(s * EPW, EPW)],
            pmax_hbm.at[pl.ds(s * NW * EPW + w * EPW, EPW)], sem).wait()
        pltpu.make_async_copy(
            accidx.at[pl.ds(s * EPW, EPW)],
            pidx_hbm.at[pl.ds(s * NW * EPW + w * EPW, EPW)], sem).wait()


_phase1 = functools.partial(
    pl.kernel,
    out_type=(
        jax.ShapeDtypeStruct((NW * ENT,), jnp.float32),
        jax.ShapeDtypeStruct((NW * ENT,), jnp.int32),
    ),
    mesh=_mesh,
    scratch_types=[
        pltpu.VMEM((RPW_PAD,), jnp.int32),
        pltpu.VMEM((2 * CHW,), jnp.float32),
        pltpu.VMEM_SHARED((NS * 2 * CHW,), jnp.float32),
        pltpu.VMEM((ENT,), jnp.float32),
        pltpu.VMEM((ENT,), jnp.int32),
        pltpu.VMEM((C,), jnp.float32),
        pltpu.VMEM((C,), jnp.int32),
        pltpu.SMEM((1,), jnp.int32),
        pltpu.SemaphoreType.DMA,
        pltpu.SemaphoreType.DMA,
        pltpu.SemaphoreType.DMA,
    ],
)(_phase1_body)


def _phase2_body(pmax_hbm, pidx_hbm, tflat_hbm, pflat_hbm, batch_hbm, out_hbm,
                 bufm, bufi, gidx, tvals, pvals, btail,
                 psum_v, outv, shared, sem):
    cid = lax.axis_index("c")
    sid = lax.axis_index("s")

    @pl.when(cid == 0)
    def _():
        base_e = sid * EPW

        pltpu.sync_copy(pmax_hbm.at[pl.ds(sid * NW * EPW, NW * EPW)], bufm)
        pltpu.sync_copy(pidx_hbm.at[pl.ds(sid * NW * EPW, NW * EPW)], bufi)

        iota = lax.iota(jnp.int32, LANES)

        def merge_entry(k, _):
            def merge_p(p, c):
                a, ai = c
                b = bufm[pl.ds(p * EPW + k * LANES, LANES)]
                bi = bufi[pl.ds(p * EPW + k * LANES, LANES)]
                m = b > a
                return (jnp.where(m, b, a), jnp.where(m, bi, ai))

            a, ai = lax.fori_loop(
                1, NW, merge_p,
                (bufm[pl.ds(k * LANES, LANES)], bufi[pl.ds(k * LANES, LANES)]))
            col = jnp.full((LANES,), lax.rem(k, CG) * LANES, jnp.int32) + iota
            gidx[pl.ds(k * LANES, LANES)] = ai * C + col
            return 0

        lax.fori_loop(0, EPW // LANES, merge_entry, 0)

        pltpu.async_copy(tflat_hbm.at[gidx], tvals, sem)
        pltpu.async_copy(pflat_hbm.at[gidx], pvals, sem)
        pltpu.make_async_copy(tflat_hbm.at[gidx], tvals, sem).wait()
        pltpu.make_async_copy(pflat_hbm.at[gidx], pvals, sem).wait()

        pltpu.sync_copy(batch_hbm.at[pl.ds(N - LANES, LANES)], btail)
        g_count = btail[...][LANES - 1] + 1

        def sum_body(k, acc):
            t = tvals[pl.ds(k * LANES, LANES)]
            p = pvals[pl.ds(k * LANES, LANES)]
            rel = jnp.abs(p - t) / (jnp.abs(t) + jnp.float32(EPS))
            evec = jnp.full((LANES,), base_e + k * LANES, jnp.int32) + iota
            gvec = lax.shift_right_logical(evec, jnp.full((LANES,), 7, jnp.int32))
            return acc + jnp.where(gvec < g_count, rel, jnp.float32(0.0))

        acc = lax.fori_loop(0, EPW // LANES, sum_body,
                            jnp.zeros((LANES,), jnp.float32))
        psum_v[...] = acc
        pltpu.sync_copy(psum_v, shared.at[pl.ds(sid * LANES, LANES)])
        plsc.subcore_barrier()

        @pl.when(sid == 0)
        def _():
            def tot_body(p, a):
                pltpu.sync_copy(shared.at[pl.ds(p * LANES, LANES)], psum_v)
                return a + psum_v[...]

            accv = lax.fori_loop(0, NS, tot_body, jnp.zeros((LANES,), jnp.float32))
            total = accv[0]
            for lane in range(1, LANES):
                total = total + accv[lane]
            denom = jnp.full((LANES,), g_count, jnp.int32).astype(jnp.float32)
            denom = denom * jnp.float32(C)
            tot_v = jnp.full((LANES,), total, jnp.float32)
            outv[...] = tot_v / denom * jnp.float32(10000.0)
            pltpu.sync_copy(outv, out_hbm)


_phase2 = functools.partial(
    pl.kernel,
    out_type=jax.ShapeDtypeStruct((LANES,), jnp.float32),
    mesh=_mesh,
    scratch_types=[
        pltpu.VMEM((NW * EPW,), jnp.float32),
        pltpu.VMEM((NW * EPW,), jnp.int32),
        pltpu.VMEM((EPW,), jnp.int32),
        pltpu.VMEM((EPW,), jnp.float32),
        pltpu.VMEM((EPW,), jnp.float32),
        pltpu.VMEM((LANES,), jnp.int32),
        pltpu.VMEM((LANES,), jnp.float32),
        pltpu.VMEM((LANES,), jnp.float32),
        pltpu.VMEM_SHARED((NS * LANES,), jnp.float32),
        pltpu.SemaphoreType.DMA,
    ],
)(_phase2_body)


def kernel(pred, target, batch, x):
    del x
    batch_i32 = batch.astype(jnp.int32)
    batch_pad = jnp.pad(
        batch_i32.reshape(NW, RPW), ((0, 0), (0, RPW_PAD - RPW))).reshape(-1)
    tflat = target.reshape(-1)
    pflat = pred.reshape(-1)
    pmax, pidx = _phase1(tflat, batch_pad)
    out = _phase2(pmax, pidx, tflat, pflat, batch_i32)
    return out[0]


# speedup vs baseline: 1.0684x; 1.0684x over previous
import functools

import jax
import jax.numpy as jnp
from jax import lax
from jax.experimental import pallas as pl
from jax.experimental.pallas import tpu as pltpu
from jax.experimental.pallas import tpu_sc as plsc

EPS = 1e-08
N = 100000
C = 128
G_MAX = 64
NC = 2
NS = 16
NW = NC * NS
RPW = N // NW
RPW_PAD = 3144
CHUNK = 125
CHW = CHUNK * C
NCHUNK = RPW // CHUNK
ENT = G_MAX * C
EPW = ENT // NS
EPH = EPW // NC
CG = C // 16
LANES = 16
NBUF = 4

_mesh = plsc.VectorSubcoreMesh(core_axis_name="c", subcore_axis_name="s")


def _phase1_body(tflat_hbm, batch_hbm, pmax_hbm, pidx_hbm,
                 batch_v, buf, accmax, accidx, regm, regi, segs, sem):
    cid = lax.axis_index("c")
    sid = lax.axis_index("s")
    w = cid * NS + sid
    row0 = w * RPW

    pltpu.sync_copy(batch_hbm.at[pl.ds(w * RPW_PAD, RPW_PAD)], batch_v)

    def init_body(k, _):
        accmax[pl.ds(k * LANES, LANES)] = jnp.full((LANES,), -1.0, jnp.float32)
        accidx[pl.ds(k * LANES, LANES)] = jnp.full((LANES,), N - 1, jnp.int32)
        return 0

    lax.fori_loop(0, ENT // LANES, init_body, 0)
    for cg in range(CG):
        regm[pl.ds(cg * LANES, LANES)] = jnp.full((LANES,), -1.0, jnp.float32)
        regi[pl.ds(cg * LANES, LANES)] = jnp.full((LANES,), N - 1, jnp.int32)
    segs[0] = batch_v[pl.ds(0, LANES)][0]

    def chunk_dma(ch):
        return pltpu.make_async_copy(
            tflat_hbm.at[pl.ds((row0 + ch * CHUNK) * C, CHW)],
            buf.at[pl.ds(lax.rem(ch, NBUF) * CHW, CHW)],
            sem)

    for pre in range(NBUF - 1):
        chunk_dma(pre).start()

    def flush(seg, vm, vi):
        base = seg * C
        for cg in range(CG):
            cur = accmax[pl.ds(base + cg * LANES, LANES)]
            curi = accidx[pl.ds(base + cg * LANES, LANES)]
            m = vm[cg] > cur
            accmax[pl.ds(base + cg * LANES, LANES)] = jnp.where(m, vm[cg], cur)
            accidx[pl.ds(base + cg * LANES, LANES)] = jnp.where(m, vi[cg], curi)

    def chunk_body(ch, _):
        chunk_dma(ch).wait()

        @pl.when(ch + NBUF - 1 < NCHUNK)
        def _():
            chunk_dma(ch + NBUF - 1).start()

        boff = lax.rem(ch, NBUF) * CHW
        cur_seg = segs[0]
        first = batch_v[pl.ds(ch * CHUNK, LANES)][0]
        last = batch_v[pl.ds(ch * CHUNK + CHUNK - LANES, LANES)][LANES - 1]
        uniform = jnp.logical_and(first == last, first == cur_seg)

        @pl.when(uniform)
        def _():
            vm = [regm[pl.ds(cg * LANES, LANES)] for cg in range(CG)]
            vi = [regi[pl.ds(cg * LANES, LANES)] for cg in range(CG)]

            def row_body(r, rc):
                rvm = list(rc[:CG])
                rvi = list(rc[CG:])
                i0 = r * 2
                i1 = r * 2 + 1
                ridx0 = jnp.full((LANES,), row0 + ch * CHUNK + i0, jnp.int32)
                ridx1 = jnp.full((LANES,), row0 + ch * CHUNK + i1, jnp.int32)
                for cg in range(CG):
                    v0 = jnp.abs(buf[pl.ds(boff + i0 * C + cg * LANES, LANES)])
                    v1 = jnp.abs(buf[pl.ds(boff + i1 * C + cg * LANES, LANES)])
                    m01 = v1 > v0
                    vp = jnp.maximum(v0, v1)
                    ip = jnp.where(m01, ridx1, ridx0)
                    m = vp > rvm[cg]
                    rvm[cg] = jnp.maximum(rvm[cg], vp)
                    rvi[cg] = jnp.where(m, ip, rvi[cg])
                return tuple(rvm + rvi)

            res = lax.fori_loop(0, CHUNK // 2, row_body, tuple(vm + vi))

            ilast = CHUNK - 1
            rvm = list(res[:CG])
            rvi = list(res[CG:])
            ridx = jnp.full((LANES,), row0 + ch * CHUNK + ilast, jnp.int32)
            for cg in range(CG):
                v = jnp.abs(buf[pl.ds(boff + ilast * C + cg * LANES, LANES)])
                m = v > rvm[cg]
                rvm[cg] = jnp.maximum(rvm[cg], v)
                rvi[cg] = jnp.where(m, ridx, rvi[cg])
            res = tuple(rvm + rvi)
            for cg in range(CG):
                regm[pl.ds(cg * LANES, LANES)] = res[cg]
                regi[pl.ds(cg * LANES, LANES)] = res[CG + cg]

        @pl.when(jnp.logical_not(uniform))
        def _():
            vm = [regm[pl.ds(cg * LANES, LANES)] for cg in range(CG)]
            vi = [regi[pl.ds(cg * LANES, LANES)] for cg in range(CG)]
            flush(cur_seg, vm, vi)

            def row_body(i, _):
                seg = batch_v[pl.ds(ch * CHUNK + i, LANES)][0]
                base = seg * C
                ridx = jnp.full((LANES,), row0 + ch * CHUNK + i, jnp.int32)
                for cg in range(CG):
                    v = jnp.abs(buf[pl.ds(boff + i * C + cg * LANES, LANES)])
                    cur = accmax[pl.ds(base + cg * LANES, LANES)]
                    curi = accidx[pl.ds(base + cg * LANES, LANES)]
                    m = v > cur
                    accmax[pl.ds(base + cg * LANES, LANES)] = jnp.where(m, v, cur)
                    accidx[pl.ds(base + cg * LANES, LANES)] = jnp.where(
                        m, ridx, curi)
                return 0

            lax.fori_loop(0, CHUNK, row_body, 0)
            nbase = last * C
            for cg in range(CG):
                regm[pl.ds(cg * LANES, LANES)] = accmax[
                    pl.ds(nbase + cg * LANES, LANES)]
                regi[pl.ds(cg * LANES, LANES)] = accidx[
                    pl.ds(nbase + cg * LANES, LANES)]
            segs[0] = last

        return 0

    lax.fori_loop(0, NCHUNK, chunk_body, 0)
    flush(segs[0],
          [regm[pl.ds(cg * LANES, LANES)] for cg in range(CG)],
          [regi[pl.ds(cg * LANES, LANES)] for cg in range(CG)])

    for s in range(NS):
        pltpu.async_copy(
            accmax.at[pl.ds(s * EPW, EPW)],
            pmax_hbm.at[pl.ds(s * NW * EPW + w * EPW, EPW)], sem)
        pltpu.async_copy(
            accidx.at[pl.ds(s * EPW, EPW)],
            pidx_hbm.at[pl.ds(s * NW * EPW + w * EPW, EPW)], sem)
    for s in range(NS):
        pltpu.make_async_copy(
            accmax.at[pl.ds(s * EPW, EPW)],
            pmax_hbm.at[pl.ds(s * NW * EPW + w * EPW, EPW)], sem).wait()
        pltpu.make_async_copy(
            accidx.at[pl.ds(s * EPW, EPW)],
            pidx_hbm.at[pl.ds(s * NW * EPW + w * EPW, EPW)], sem).wait()


_phase1 = functools.partial(
    pl.kernel,
    out_type=(
        jax.ShapeDtypeStruct((NW * ENT,), jnp.float32),
        jax.ShapeDtypeStruct((NW * ENT,), jnp.int32),
    ),
    mesh=_mesh,
    scratch_types=[
        pltpu.VMEM((RPW_PAD,), jnp.int32),
        pltpu.VMEM((NBUF * CHW,), jnp.float32),
        pltpu.VMEM((ENT,), jnp.float32),
        pltpu.VMEM((ENT,), jnp.int32),
        pltpu.VMEM((C,), jnp.float32),
        pltpu.VMEM((C,), jnp.int32),
        pltpu.SMEM((1,), jnp.int32),
        pltpu.SemaphoreType.DMA,
    ],
)(_phase1_body)


def _phase2_body(pmax_hbm, pidx_hbm, tflat_hbm, pflat_hbm, batch_hbm, out_hbm,
                 bufm, bufi, gidx, tvals, pvals, btail,
                 psum_v, outv, shared, sem, semg):
    cid = lax.axis_index("c")
    sid = lax.axis_index("s")
    half = cid * EPH
    base_e = sid * EPW + half

    pltpu.async_copy(pmax_hbm.at[pl.ds(sid * NW * EPW, NW * EPW)], bufm, sem)
    pltpu.async_copy(pidx_hbm.at[pl.ds(sid * NW * EPW, NW * EPW)], bufi, sem)
    pltpu.async_copy(batch_hbm.at[pl.ds(N - LANES, LANES)], btail, sem)
    pltpu.make_async_copy(pmax_hbm.at[pl.ds(sid * NW * EPW, NW * EPW)],
                          bufm, sem).wait()
    pltpu.make_async_copy(pidx_hbm.at[pl.ds(sid * NW * EPW, NW * EPW)],
                          bufi, sem).wait()
    pltpu.make_async_copy(batch_hbm.at[pl.ds(N - LANES, LANES)],
                          btail, sem).wait()

    iota = lax.iota(jnp.int32, LANES)

    def merge_entry(k, _):
        off = half + k * LANES

        def merge_p(p, c):
            a, ai = c
            b = bufm[pl.ds(p * EPW + off, LANES)]
            bi = bufi[pl.ds(p * EPW + off, LANES)]
            m = b > a
            return (jnp.where(m, b, a), jnp.where(m, bi, ai))

        a, ai = lax.fori_loop(
            1, NW, merge_p,
            (bufm[pl.ds(off, LANES)], bufi[pl.ds(off, LANES)]))
        col = jnp.full(
            (LANES,), lax.rem(lax.rem(base_e, C) + k * LANES, C),
            jnp.int32) + iota
        gidx[pl.ds(k * LANES, LANES)] = ai * C + col
        return 0

    lax.fori_loop(0, EPH // LANES, merge_entry, 0)

    pltpu.async_copy(tflat_hbm.at[gidx], tvals, semg)
    pltpu.async_copy(pflat_hbm.at[gidx], pvals, semg)
    pltpu.make_async_copy(tflat_hbm.at[gidx], tvals, semg).wait()
    pltpu.make_async_copy(pflat_hbm.at[gidx], pvals, semg).wait()

    g_count = btail[...][LANES - 1] + 1

    def sum_body(k, acc):
        t = tvals[pl.ds(k * LANES, LANES)]
        p = pvals[pl.ds(k * LANES, LANES)]
        rel = jnp.abs(p - t) / (jnp.abs(t) + jnp.float32(EPS))
        evec = jnp.full((LANES,), base_e + k * LANES, jnp.int32) + iota
        gvec = lax.shift_right_logical(evec, jnp.full((LANES,), 7, jnp.int32))
        return acc + jnp.where(gvec < g_count, rel, jnp.float32(0.0))

    acc = lax.fori_loop(0, EPH // LANES, sum_body,
                        jnp.zeros((LANES,), jnp.float32))
    psum_v[...] = acc
    pltpu.sync_copy(psum_v, shared.at[pl.ds(sid * LANES, LANES)])
    plsc.subcore_barrier()

    @pl.when(sid == 0)
    def _():
        def tot_body(p, a):
            pltpu.sync_copy(shared.at[pl.ds(p * LANES, LANES)], psum_v)
            return a + psum_v[...]

        accv = lax.fori_loop(0, NS, tot_body, jnp.zeros((LANES,), jnp.float32))
        total = accv[0]
        for lane in range(1, LANES):
            total = total + accv[lane]
        denom = jnp.full((LANES,), g_count, jnp.int32).astype(jnp.float32)
        denom = denom * jnp.float32(C)
        tot_v = jnp.full((LANES,), total, jnp.float32)
        outv[...] = tot_v / denom * jnp.float32(10000.0)
        pltpu.sync_copy(outv, out_hbm.at[pl.ds(cid * LANES, LANES)])


_phase2 = functools.partial(
    pl.kernel,
    out_type=jax.ShapeDtypeStruct((NC * LANES,), jnp.float32),
    mesh=_mesh,
    scratch_types=[
        pltpu.VMEM((NW * EPW,), jnp.float32),
        pltpu.VMEM((NW * EPW,), jnp.int32),
        pltpu.VMEM((EPH,), jnp.int32),
        pltpu.VMEM((EPH,), jnp.float32),
        pltpu.VMEM((EPH,), jnp.float32),
        pltpu.VMEM((LANES,), jnp.int32),
        pltpu.VMEM((LANES,), jnp.float32),
        pltpu.VMEM((LANES,), jnp.float32),
        pltpu.VMEM_SHARED((NS * LANES,), jnp.float32),
        pltpu.SemaphoreType.DMA,
        pltpu.SemaphoreType.DMA,
    ],
)(_phase2_body)


def kernel(pred, target, batch, x):
    del x
    batch_i32 = batch.astype(jnp.int32)
    batch_pad = jnp.pad(
        batch_i32.reshape(NW, RPW), ((0, 0), (0, RPW_PAD - RPW))).reshape(-1)
    tflat = target.reshape(-1)
    pflat = pred.reshape(-1)
    pmax, pidx = _phase1(tflat, batch_pad)
    out = _phase2(pmax, pidx, tflat, pflat, batch_i32)
    return out[0] + out[LANES]
